# Initial kernel scaffold; baseline (speedup 1.0000x reference)
import jax
import jax.numpy as jnp
from jax.experimental import pallas as pl


def kernel(features, edge_indices, edge_weights, W1, b1, W2, b2, Wc1, bc1, Wc2, bc2):
    raise NotImplementedError("write your pallas kernel here")


# speedup vs baseline: 2.2039x; 2.2039x over previous
import jax
import jax.numpy as jnp
from jax import lax
from jax.experimental import pallas as pl
from jax.experimental.pallas import tpu as pltpu
from jax.experimental.pallas import tpu_sc as plsc

N = 10000
NP = 10240
D = 128
E = 320000
NC = 2
NS = 16
NRANGE = NP // NC
NTRASH = 1024
NACC = NRANGE + NTRASH
CHUNK = 128
NCHUNK = 160
EPS = NCHUNK * CHUNK
EPAD = NS * EPS
ZROWS = NACC // NS
ORAW = NRANGE // NS


def _make_sc_aggregate(with_deg: bool):
  mesh = plsc.VectorSubcoreMesh(
      core_axis_name="c", subcore_axis_name="s", num_cores=NC, num_subcores=NS)
  out_type = [jax.ShapeDtypeStruct((NP, D), jnp.float32)]
  scratch = [
      pltpu.VMEM((NCHUNK, CHUNK), jnp.int32),
      pltpu.VMEM((NCHUNK, CHUNK), jnp.int32),
      pltpu.VMEM((CHUNK, D), jnp.float32),
      pltpu.VMEM((128, D), jnp.float32),
      pltpu.VMEM_SHARED((NACC, D), jnp.float32),
      pltpu.SemaphoreType.DMA,
  ]
  if with_deg:
    out_type.append(jax.ShapeDtypeStruct((NP,), jnp.float32))
    scratch += [
        pltpu.VMEM((CHUNK,), jnp.float32),
        pltpu.VMEM((ZROWS,), jnp.float32),
        pltpu.VMEM_SHARED((NACC,), jnp.float32),
    ]

  def body(x_hbm, src_hbm, dst_hbm, *rest):
    if with_deg:
      (out_hbm, deg_hbm, idx_s, idx_d, rows, zbuf, acc, sem,
       ones_v, dz, dacc) = rest
    else:
      out_hbm, idx_s, idx_d, rows, zbuf, acc, sem = rest
    c = lax.axis_index("c")
    s = lax.axis_index("s")
    base = c * NRANGE

    def zrow(i, _):
      def zcol(j, _):
        zbuf[i, pl.ds(j * 16, 16)] = jnp.zeros((16,), jnp.float32)
        return 0
      return lax.fori_loop(0, D // 16, zcol, 0)
    lax.fori_loop(0, 128, zrow, 0)
    for k in range(ZROWS // 128):
      pltpu.sync_copy(zbuf, acc.at[pl.ds(s * ZROWS + k * 128, 128)])
    if with_deg:
      for k in range(CHUNK // 16):
        ones_v[pl.ds(k * 16, 16)] = jnp.ones((16,), jnp.float32)
      for k in range(ZROWS // 16):
        dz[pl.ds(k * 16, 16)] = jnp.zeros((16,), jnp.float32)
      pltpu.sync_copy(dz, dacc.at[pl.ds(s * ZROWS, ZROWS)])

    pltpu.sync_copy(src_hbm.at[s], idx_s)
    pltpu.sync_copy(dst_hbm.at[s], idx_d)
    def remap_row(i, _):
      def remap_col(j, _):
        v = idx_d[i, pl.ds(j * 16, 16)]
        t = v - base
        ok = (t >= 0) & (t < NRANGE)
        trash = NRANGE + (v & (NTRASH - 1))
        idx_d[i, pl.ds(j * 16, 16)] = jnp.where(ok, t, trash)
        return 0
      return lax.fori_loop(0, CHUNK // 16, remap_col, 0)
    lax.fori_loop(0, NCHUNK, remap_row, 0)
    plsc.subcore_barrier()

    def chunk(j, _):
      pltpu.async_copy(x_hbm.at[idx_s.at[j]], rows, sem).wait()
      pltpu.sync_copy(rows, acc.at[idx_d.at[j]], add=True)
      if with_deg:
        pltpu.sync_copy(ones_v, dacc.at[idx_d.at[j]], add=True)
      return 0
    lax.fori_loop(0, NCHUNK, chunk, 0)
    plsc.subcore_barrier()

    for k in range(ORAW // 128):
      pltpu.sync_copy(acc.at[pl.ds(s * ORAW + k * 128, 128)], zbuf)
      pltpu.sync_copy(zbuf, out_hbm.at[pl.ds(base + s * ORAW + k * 128, 128)])
    rem = ORAW % 128
    if rem:
      off = ORAW - rem
      pltpu.sync_copy(acc.at[pl.ds(s * ORAW + off, rem)], zbuf.at[pl.ds(0, rem)])
      pltpu.sync_copy(zbuf.at[pl.ds(0, rem)],
                      out_hbm.at[pl.ds(base + s * ORAW + off, rem)])
    if with_deg:
      pltpu.sync_copy(dacc.at[pl.ds(s * ORAW, ORAW)], dz.at[pl.ds(0, ORAW)])
      pltpu.sync_copy(dz.at[pl.ds(0, ORAW)],
                      deg_hbm.at[pl.ds(base + s * ORAW, ORAW)])

  return pl.kernel(body, out_type=out_type, mesh=mesh, scratch_types=scratch)


_sc_aggregate_deg = _make_sc_aggregate(True)
_sc_aggregate = _make_sc_aggregate(False)

BS = 1024


def _tc_layer1(x_ref, a_ref, d_ref, wx_ref, wn_ref, b_ref, o_ref):
  inv = 1.0 / jnp.maximum(d_ref[...], 1.0)
  o_ref[...] = jnp.maximum(
      jnp.dot(x_ref[...], wx_ref[...], preferred_element_type=jnp.float32)
      + jnp.dot(a_ref[...] * inv, wn_ref[...],
                preferred_element_type=jnp.float32)
      + b_ref[...], 0.0)


def _tc_layer2(x_ref, a_ref, d_ref, wx_ref, wn_ref, b_ref,
               wc1_ref, bc1_ref, wc2_ref, bc2_ref, o_ref):
  inv = 1.0 / jnp.maximum(d_ref[...], 1.0)
  x2 = jnp.maximum(
      jnp.dot(x_ref[...], wx_ref[...], preferred_element_type=jnp.float32)
      + jnp.dot(a_ref[...] * inv, wn_ref[...],
                preferred_element_type=jnp.float32)
      + b_ref[...], 0.0)
  h = jnp.maximum(
      jnp.dot(x2, wc1_ref[...], preferred_element_type=jnp.float32)
      + bc1_ref[...], 0.0)
  o_ref[...] = (
      jnp.dot(h, wc2_ref[...], preferred_element_type=jnp.float32)
      + bc2_ref[...])


def _row_spec(w):
  return pl.BlockSpec((BS, w), lambda i: (i, 0))


def _full_spec(shape):
  nd = len(shape)
  return pl.BlockSpec(shape, lambda i, _nd=nd: (0,) * nd)


def _tc1_call(x, agg, d, wx, wn, b):
  return pl.pallas_call(
      _tc_layer1,
      grid=(NP // BS,),
      in_specs=[
          _row_spec(D), _row_spec(D), _row_spec(1),
          _full_spec((D, D)), _full_spec((D, D)), _full_spec((1, D)),
      ],
      out_specs=_row_spec(D),
      out_shape=jax.ShapeDtypeStruct((NP, D), jnp.float32),
  )(x, agg, d, wx, wn, b)


def _tc2_call(x, agg, d, wx, wn, b, wc1, bc1, wc2, bc2):
  return pl.pallas_call(
      _tc_layer2,
      grid=(NP // BS,),
      in_specs=[
          _row_spec(D), _row_spec(D), _row_spec(1),
          _full_spec((D, D)), _full_spec((D, D)), _full_spec((1, D)),
          _full_spec((D, D // 2)), _full_spec((1, D // 2)),
          _full_spec((D // 2, 2)), _full_spec((1, 2)),
      ],
      out_specs=_row_spec(2),
      out_shape=jax.ShapeDtypeStruct((NP, 2), jnp.float32),
  )(x, agg, d, wx, wn, b, wc1, bc1, wc2, bc2)


@jax.jit
def kernel(features, edge_indices, edge_weights, W1, b1, W2, b2,
           Wc1, bc1, Wc2, bc2):
  del edge_weights
  x0 = jnp.pad(features, ((0, NP - N), (0, 0)))
  src3 = jnp.pad(edge_indices[0, 0], (0, EPAD - E)).reshape(NS, NCHUNK, CHUNK)
  dst3 = jnp.pad(edge_indices[0, 1], (0, EPAD - E),
                 constant_values=N).reshape(NS, NCHUNK, CHUNK)
  w1x = W1[:, :D].T
  w1n = W1[:, D:].T
  w2x = W2[:, :D].T
  w2n = W2[:, D:].T

  agg1, deg = _sc_aggregate_deg(x0, src3, dst3)
  d = deg.reshape(NP, 1)
  x1 = _tc1_call(x0, agg1, d, w1x, w1n, b1.reshape(1, D))

  (agg2,) = _sc_aggregate(x1, src3, dst3)
  logits = _tc2_call(x1, agg2, d, w2x, w2n, b2.reshape(1, D),
                     Wc1.T, bc1.reshape(1, D // 2),
                     Wc2.T, bc2.reshape(1, 2))
  return logits[:N]


# speedup vs baseline: 2.4461x; 1.1099x over previous
import jax
import jax.numpy as jnp
from jax import lax
from jax.experimental import pallas as pl
from jax.experimental.pallas import tpu as pltpu
from jax.experimental.pallas import tpu_sc as plsc

N = 10000
NP = 10240
D = 128
E = 320000
NC = 2
NS = 16
NRANGE = NP // NC
NTRASH = 1024
NACC = NRANGE + NTRASH
CHUNK = 128
NCHUNK = 160
EPS = NCHUNK * CHUNK
EPAD = NS * EPS
ZROWS = NACC // NS
ORAW = NRANGE // NS
ZB = 32


def _make_sc_aggregate(with_deg: bool):
  mesh = plsc.VectorSubcoreMesh(
      core_axis_name="c", subcore_axis_name="s", num_cores=NC, num_subcores=NS)
  out_type = [jax.ShapeDtypeStruct((NP, D), jnp.float32)]
  scratch = [
      pltpu.VMEM((NCHUNK, CHUNK), jnp.int32),
      pltpu.VMEM((NCHUNK, CHUNK), jnp.int32),
      pltpu.VMEM((CHUNK, D), jnp.float32),
      pltpu.VMEM((CHUNK, D), jnp.float32),
      pltpu.VMEM((ZB, D), jnp.float32),
      pltpu.VMEM_SHARED((NACC, D), jnp.float32),
      pltpu.SemaphoreType.DMA,
      pltpu.SemaphoreType.DMA,
      pltpu.SemaphoreType.DMA,
      pltpu.SemaphoreType.DMA,
  ]
  if with_deg:
    out_type.append(jax.ShapeDtypeStruct((NP,), jnp.float32))
    scratch += [
        pltpu.VMEM((CHUNK,), jnp.float32),
        pltpu.VMEM((ZROWS,), jnp.float32),
        pltpu.VMEM_SHARED((NACC,), jnp.float32),
        pltpu.SemaphoreType.DMA,
        pltpu.SemaphoreType.DMA,
    ]

  def body(x_hbm, src_hbm, dst_hbm, *rest):
    if with_deg:
      (out_hbm, deg_hbm, idx_s, idx_d, rows0, rows1, zbuf, acc,
       sg0, sg1, ss0, ss1, ones_v, dz, dacc, sd0, sd1) = rest
    else:
      (out_hbm, idx_s, idx_d, rows0, rows1, zbuf, acc,
       sg0, sg1, ss0, ss1) = rest
    c = lax.axis_index("c")
    s = lax.axis_index("s")
    base = c * NRANGE

    def zrow(i, _):
      def zcol(j, _):
        zbuf[i, pl.ds(j * 16, 16)] = jnp.zeros((16,), jnp.float32)
        return 0
      return lax.fori_loop(0, D // 16, zcol, 0)
    lax.fori_loop(0, ZB, zrow, 0)
    for k in range(ZROWS // ZB):
      pltpu.sync_copy(zbuf, acc.at[pl.ds(s * ZROWS + k * ZB, ZB)])
    if with_deg:
      for k in range(CHUNK // 16):
        ones_v[pl.ds(k * 16, 16)] = jnp.ones((16,), jnp.float32)
      for k in range(ZROWS // 16):
        dz[pl.ds(k * 16, 16)] = jnp.zeros((16,), jnp.float32)
      pltpu.sync_copy(dz, dacc.at[pl.ds(s * ZROWS, ZROWS)])

    pltpu.sync_copy(src_hbm.at[s], idx_s)
    pltpu.sync_copy(dst_hbm.at[s], idx_d)
    def remap_row(i, _):
      def remap_col(j, _):
        v = idx_d[i, pl.ds(j * 16, 16)]
        t = v - base
        ok = (t >= 0) & (t < NRANGE)
        trash = NRANGE + (v & (NTRASH - 1))
        idx_d[i, pl.ds(j * 16, 16)] = jnp.where(ok, t, trash)
        return 0
      return lax.fori_loop(0, CHUNK // 16, remap_col, 0)
    lax.fori_loop(0, NCHUNK, remap_row, 0)
    plsc.subcore_barrier()

    def g_issue(j, rows, sem):
      pltpu.async_copy(x_hbm.at[idx_s.at[j]], rows, sem)
    def g_wait(rows, sem):
      pltpu.make_async_copy(x_hbm.at[pl.ds(0, CHUNK)], rows, sem).wait()
    def s_issue(j, rows, sem):
      pltpu.async_copy(rows, acc.at[idx_d.at[j]], sem, add=True)
    def s_wait(rows, sem):
      pltpu.make_async_copy(x_hbm.at[pl.ds(0, CHUNK)], rows, sem).wait()
    if with_deg:
      def d_issue(j, sem):
        pltpu.async_copy(ones_v, dacc.at[idx_d.at[j]], sem, add=True)
      def d_wait(sem):
        pltpu.make_async_copy(deg_hbm.at[pl.ds(0, CHUNK)], ones_v, sem).wait()

    g_issue(0, rows0, sg0)
    def pair(p, _):
      j0 = 2 * p
      j1 = j0 + 1
      @pl.when(p > 0)
      def _():
        s_wait(rows1, ss1)
      g_issue(j1, rows1, sg1)
      g_wait(rows0, sg0)
      s_issue(j0, rows0, ss0)
      if with_deg:
        @pl.when(p > 0)
        def _():
          d_wait(sd0)
        d_issue(j0, sd0)
      @pl.when(p < NCHUNK // 2 - 1)
      def _():
        s_wait(rows0, ss0)
        g_issue(j0 + 2, rows0, sg0)
      g_wait(rows1, sg1)
      s_issue(j1, rows1, ss1)
      if with_deg:
        @pl.when(p > 0)
        def _():
          d_wait(sd1)
        d_issue(j1, sd1)
      return 0
    lax.fori_loop(0, NCHUNK // 2, pair, 0)
    s_wait(rows0, ss0)
    s_wait(rows1, ss1)
    if with_deg:
      d_wait(sd0)
      d_wait(sd1)
    plsc.subcore_barrier()

    for k in range(ORAW // ZB):
      pltpu.sync_copy(acc.at[pl.ds(s * ORAW + k * ZB, ZB)], zbuf)
      pltpu.sync_copy(zbuf, out_hbm.at[pl.ds(base + s * ORAW + k * ZB, ZB)])
    if with_deg:
      pltpu.sync_copy(dacc.at[pl.ds(s * ORAW, ORAW)], dz.at[pl.ds(0, ORAW)])
      pltpu.sync_copy(dz.at[pl.ds(0, ORAW)],
                      deg_hbm.at[pl.ds(base + s * ORAW, ORAW)])

  return pl.kernel(body, out_type=out_type, mesh=mesh, scratch_types=scratch)


_sc_aggregate_deg = _make_sc_aggregate(True)
_sc_aggregate = _make_sc_aggregate(False)

BS = 1024


def _tc_layer1(x_ref, a_ref, d_ref, wx_ref, wn_ref, b_ref, o_ref):
  inv = 1.0 / jnp.maximum(d_ref[...], 1.0)
  o_ref[...] = jnp.maximum(
      jnp.dot(x_ref[...], wx_ref[...], preferred_element_type=jnp.float32)
      + jnp.dot(a_ref[...] * inv, wn_ref[...],
                preferred_element_type=jnp.float32)
      + b_ref[...], 0.0)


def _tc_layer2(x_ref, a_ref, d_ref, wx_ref, wn_ref, b_ref,
               wc1_ref, bc1_ref, wc2_ref, bc2_ref, o_ref):
  inv = 1.0 / jnp.maximum(d_ref[...], 1.0)
  x2 = jnp.maximum(
      jnp.dot(x_ref[...], wx_ref[...], preferred_element_type=jnp.float32)
      + jnp.dot(a_ref[...] * inv, wn_ref[...],
                preferred_element_type=jnp.float32)
      + b_ref[...], 0.0)
  h = jnp.maximum(
      jnp.dot(x2, wc1_ref[...], preferred_element_type=jnp.float32)
      + bc1_ref[...], 0.0)
  o_ref[...] = (
      jnp.dot(h, wc2_ref[...], preferred_element_type=jnp.float32)
      + bc2_ref[...])


def _row_spec(w):
  return pl.BlockSpec((BS, w), lambda i: (i, 0))


def _full_spec(shape):
  nd = len(shape)
  return pl.BlockSpec(shape, lambda i, _nd=nd: (0,) * nd)


def _tc1_call(x, agg, d, wx, wn, b):
  return pl.pallas_call(
      _tc_layer1,
      grid=(NP // BS,),
      in_specs=[
          _row_spec(D), _row_spec(D), _row_spec(1),
          _full_spec((D, D)), _full_spec((D, D)), _full_spec((1, D)),
      ],
      out_specs=_row_spec(D),
      out_shape=jax.ShapeDtypeStruct((NP, D), jnp.float32),
  )(x, agg, d, wx, wn, b)


def _tc2_call(x, agg, d, wx, wn, b, wc1, bc1, wc2, bc2):
  return pl.pallas_call(
      _tc_layer2,
      grid=(NP // BS,),
      in_specs=[
          _row_spec(D), _row_spec(D), _row_spec(1),
          _full_spec((D, D)), _full_spec((D, D)), _full_spec((1, D)),
          _full_spec((D, D // 2)), _full_spec((1, D // 2)),
          _full_spec((D // 2, 2)), _full_spec((1, 2)),
      ],
      out_specs=_row_spec(2),
      out_shape=jax.ShapeDtypeStruct((NP, 2), jnp.float32),
  )(x, agg, d, wx, wn, b, wc1, bc1, wc2, bc2)


@jax.jit
def kernel(features, edge_indices, edge_weights, W1, b1, W2, b2,
           Wc1, bc1, Wc2, bc2):
  del edge_weights
  x0 = jnp.pad(features, ((0, NP - N), (0, 0)))
  src3 = jnp.pad(edge_indices[0, 0], (0, EPAD - E)).reshape(NS, NCHUNK, CHUNK)
  dst3 = jnp.pad(edge_indices[0, 1], (0, EPAD - E),
                 constant_values=N).reshape(NS, NCHUNK, CHUNK)
  w1x = W1[:, :D].T
  w1n = W1[:, D:].T
  w2x = W2[:, :D].T
  w2n = W2[:, D:].T

  agg1, deg = _sc_aggregate_deg(x0, src3, dst3)
  d = deg.reshape(NP, 1)
  x1 = _tc1_call(x0, agg1, d, w1x, w1n, b1.reshape(1, D))

  (agg2,) = _sc_aggregate(x1, src3, dst3)
  logits = _tc2_call(x1, agg2, d, w2x, w2n, b2.reshape(1, D),
                     Wc1.T, bc1.reshape(1, D // 2),
                     Wc2.T, bc2.reshape(1, 2))
  return logits[:N]


# speedup vs baseline: 2.4785x; 1.0133x over previous
import jax
import jax.numpy as jnp
from jax import lax
from jax.experimental import pallas as pl
from jax.experimental.pallas import tpu as pltpu
from jax.experimental.pallas import tpu_sc as plsc

N = 10000
NP = 10240
D = 128
E = 320000
NC = 2
NS = 16
NRANGE = NP // NC
NTRASH = 512
NACC = NRANGE + NTRASH
CHUNK = 128
NBUF = 2
LOOK = 1
EPS = 20480
FILL = NBUF * CHUNK
EPS2 = EPS + FILL + 16
DUMP = EPS + FILL
EPAD = NS * EPS
ZROWS = NACC // NS
ORAW = NRANGE // NS
ZB = 32


def _make_sc_aggregate(with_deg: bool):
  mesh = plsc.VectorSubcoreMesh(
      core_axis_name="c", subcore_axis_name="s", num_cores=NC, num_subcores=NS)
  out_type = [jax.ShapeDtypeStruct((NP, D), jnp.float32)]
  scratch = [
      pltpu.VMEM((EPS2,), jnp.int32),
      pltpu.VMEM((EPS2,), jnp.int32),
  ] + [pltpu.VMEM((CHUNK, D), jnp.float32) for _ in range(NBUF)] + [
      pltpu.VMEM((ZB, D), jnp.float32),
      pltpu.VMEM_SHARED((NACC, D), jnp.float32),
  ] + [pltpu.SemaphoreType.DMA] * (2 * NBUF)
  if with_deg:
    out_type.append(jax.ShapeDtypeStruct((NP,), jnp.float32))
    scratch += [
        pltpu.VMEM((CHUNK,), jnp.float32),
        pltpu.VMEM((ZROWS,), jnp.float32),
        pltpu.VMEM_SHARED((NACC,), jnp.float32),
    ] + [pltpu.SemaphoreType.DMA] * NBUF

  def body(x_hbm, src_hbm, dst_hbm, *rest):
    if with_deg:
      out_hbm, deg_hbm, r = rest[0], rest[1], rest[2:]
    else:
      out_hbm, r = rest[0], rest[1:]
    idx_s, idx_d = r[0], r[1]
    rows = r[2:2 + NBUF]
    zbuf, acc = r[2 + NBUF], r[3 + NBUF]
    sg = r[4 + NBUF:4 + 2 * NBUF]
    ss = r[4 + 2 * NBUF:4 + 3 * NBUF]
    if with_deg:
      ones_v, dz, dacc = r[4 + 3 * NBUF:7 + 3 * NBUF]
      sd = r[7 + 3 * NBUF:7 + 4 * NBUF]
    c = lax.axis_index("c")
    s = lax.axis_index("s")
    base = c * NRANGE

    def zrow(i, _):
      def zcol(j, _):
        zbuf[i, pl.ds(j * 16, 16)] = jnp.zeros((16,), jnp.float32)
        return 0
      return lax.fori_loop(0, D // 16, zcol, 0)
    lax.fori_loop(0, ZB, zrow, 0)
    for k in range(ZROWS // ZB):
      pltpu.sync_copy(zbuf, acc.at[pl.ds(s * ZROWS + k * ZB, ZB)])
    if with_deg:
      for k in range(CHUNK // 16):
        ones_v[pl.ds(k * 16, 16)] = jnp.ones((16,), jnp.float32)
      for k in range(ZROWS // 16):
        dz[pl.ds(k * 16, 16)] = jnp.zeros((16,), jnp.float32)
      pltpu.sync_copy(dz, dacc.at[pl.ds(s * ZROWS, ZROWS)])

    pltpu.sync_copy(src_hbm.at[s], idx_s.at[pl.ds(0, EPS)])
    pltpu.sync_copy(dst_hbm.at[s], idx_d.at[pl.ds(0, EPS)])
    lane = lax.iota(jnp.int32, 16)
    def compact(i, cnt):
      vd = idx_d[pl.ds(i * 16, 16)]
      vs = idx_s[pl.ds(i * 16, 16)]
      t = vd - base
      ok = (t >= 0) & (t < NRANGE)
      cum = plsc.cumsum(ok.astype(jnp.int32))
      pos = jnp.where(ok, cnt + cum - 1, DUMP + lane)
      plsc.store_scatter(idx_d, [pos], t)
      plsc.store_scatter(idx_s, [pos], vs)
      return cnt + jnp.max(cum)
    kept = lax.fori_loop(0, EPS // 16, compact, jnp.int32(0))
    for k in range(FILL // 16):
      pos = kept + k * 16 + lane
      plsc.store_scatter(idx_d, [pos], NRANGE + ((k * 16 + lane) & (NTRASH - 1)))
      plsc.store_scatter(idx_s, [pos], jnp.zeros((16,), jnp.int32))
    nch = kept // CHUNK + NBUF
    plsc.subcore_barrier()

    def g_issue(j, b):
      pltpu.async_copy(x_hbm.at[idx_s.at[pl.ds(j * CHUNK, CHUNK)]],
                       rows[b], sg[b])
    def g_wait(b):
      pltpu.make_async_copy(x_hbm.at[pl.ds(0, CHUNK)], rows[b], sg[b]).wait()
    def s_issue(j, b):
      pltpu.async_copy(rows[b], acc.at[idx_d.at[pl.ds(j * CHUNK, CHUNK)]],
                       ss[b], add=True)
    def s_wait(b):
      pltpu.make_async_copy(x_hbm.at[pl.ds(0, CHUNK)], rows[b], ss[b]).wait()
    if with_deg:
      def d_issue(j, b):
        pltpu.async_copy(ones_v, dacc.at[idx_d.at[pl.ds(j * CHUNK, CHUNK)]],
                         sd[b], add=True)
      def d_wait(b):
        pltpu.make_async_copy(deg_hbm.at[pl.ds(0, CHUNK)], ones_v, sd[b]).wait()

    for i in range(LOOK):
      g_issue(i, i % NBUF)
    def step(j, b):
      bL = (b + LOOK) % NBUF
      @pl.when(j + LOOK < nch)
      def _():
        @pl.when(j >= NBUF - LOOK)
        def _():
          s_wait(bL)
        g_issue(j + LOOK, bL)
      g_wait(b)
      s_issue(j, b)
      if with_deg:
        @pl.when(j >= NBUF)
        def _():
          d_wait(b)
        d_issue(j, b)
    def chunk_body(j, _):
      @pl.when(j < nch)
      def _():
        for b in range(NBUF):
          @pl.when(j % NBUF == b)
          def _(b=b):
            step(j, b)
      return 0
    lax.fori_loop(0, EPS2 // CHUNK, chunk_body, 0)
    for b in range(NBUF):
      s_wait(b)
      if with_deg:
        d_wait(b)
    plsc.subcore_barrier()

    for k in range(ORAW // ZB):
      pltpu.sync_copy(acc.at[pl.ds(s * ORAW + k * ZB, ZB)], zbuf)
      pltpu.sync_copy(zbuf, out_hbm.at[pl.ds(base + s * ORAW + k * ZB, ZB)])
    if with_deg:
      pltpu.sync_copy(dacc.at[pl.ds(s * ORAW, ORAW)], dz.at[pl.ds(0, ORAW)])
      pltpu.sync_copy(dz.at[pl.ds(0, ORAW)],
                      deg_hbm.at[pl.ds(base + s * ORAW, ORAW)])

  return pl.kernel(
      body, out_type=out_type, mesh=mesh, scratch_types=scratch,
      compiler_params=pltpu.CompilerParams(needs_layout_passes=False))


_sc_aggregate_deg = _make_sc_aggregate(True)
_sc_aggregate = _make_sc_aggregate(False)

BS = 1024


def _tc_layer1(x_ref, a_ref, d_ref, wx_ref, wn_ref, b_ref, o_ref):
  inv = 1.0 / jnp.maximum(d_ref[...], 1.0)
  o_ref[...] = jnp.maximum(
      jnp.dot(x_ref[...], wx_ref[...], preferred_element_type=jnp.float32)
      + jnp.dot(a_ref[...] * inv, wn_ref[...],
                preferred_element_type=jnp.float32)
      + b_ref[...], 0.0)


def _tc_layer2(x_ref, a_ref, d_ref, wx_ref, wn_ref, b_ref,
               wc1_ref, bc1_ref, wc2_ref, bc2_ref, o_ref):
  inv = 1.0 / jnp.maximum(d_ref[...], 1.0)
  x2 = jnp.maximum(
      jnp.dot(x_ref[...], wx_ref[...], preferred_element_type=jnp.float32)
      + jnp.dot(a_ref[...] * inv, wn_ref[...],
                preferred_element_type=jnp.float32)
      + b_ref[...], 0.0)
  h = jnp.maximum(
      jnp.dot(x2, wc1_ref[...], preferred_element_type=jnp.float32)
      + bc1_ref[...], 0.0)
  o_ref[...] = (
      jnp.dot(h, wc2_ref[...], preferred_element_type=jnp.float32)
      + bc2_ref[...])


def _row_spec(w):
  return pl.BlockSpec((BS, w), lambda i: (i, 0))


def _full_spec(shape):
  nd = len(shape)
  return pl.BlockSpec(shape, lambda i, _nd=nd: (0,) * nd)


def _tc1_call(x, agg, d, wx, wn, b):
  return pl.pallas_call(
      _tc_layer1,
      grid=(NP // BS,),
      in_specs=[
          _row_spec(D), _row_spec(D), _row_spec(1),
          _full_spec((D, D)), _full_spec((D, D)), _full_spec((1, D)),
      ],
      out_specs=_row_spec(D),
      out_shape=jax.ShapeDtypeStruct((NP, D), jnp.float32),
  )(x, agg, d, wx, wn, b)


def _tc2_call(x, agg, d, wx, wn, b, wc1, bc1, wc2, bc2):
  return pl.pallas_call(
      _tc_layer2,
      grid=(NP // BS,),
      in_specs=[
          _row_spec(D), _row_spec(D), _row_spec(1),
          _full_spec((D, D)), _full_spec((D, D)), _full_spec((1, D)),
          _full_spec((D, D // 2)), _full_spec((1, D // 2)),
          _full_spec((D // 2, 2)), _full_spec((1, 2)),
      ],
      out_specs=_row_spec(2),
      out_shape=jax.ShapeDtypeStruct((NP, 2), jnp.float32),
  )(x, agg, d, wx, wn, b, wc1, bc1, wc2, bc2)


@jax.jit
def kernel(features, edge_indices, edge_weights, W1, b1, W2, b2,
           Wc1, bc1, Wc2, bc2):
  del edge_weights
  x0 = jnp.pad(features, ((0, NP - N), (0, 0)))
  src3 = jnp.pad(edge_indices[0, 0], (0, EPAD - E)).reshape(NS, EPS)
  pad_dst = N + (jnp.arange(EPAD - E, dtype=jnp.int32) % (NP - N))
  dst3 = jnp.concatenate([edge_indices[0, 1], pad_dst]).reshape(NS, EPS)
  w1x = W1[:, :D].T
  w1n = W1[:, D:].T
  w2x = W2[:, :D].T
  w2n = W2[:, D:].T

  agg1, deg = _sc_aggregate_deg(x0, src3, dst3)
  d = deg.reshape(NP, 1)
  x1 = _tc1_call(x0, agg1, d, w1x, w1n, b1.reshape(1, D))

  (agg2,) = _sc_aggregate(x1, src3, dst3)
  logits = _tc2_call(x1, agg2, d, w2x, w2n, b2.reshape(1, D),
                     Wc1.T, bc1.reshape(1, D // 2),
                     Wc2.T, bc2.reshape(1, 2))
  return logits[:N]
